# baseline (speedup 1.0000x reference)
import jax
import jax.numpy as jnp
from jax.experimental import pallas as pl
from jax.experimental.pallas import tpu as pltpu

_NUM_CODES = 1024
_EMB = 64
_S_TILE = 4096
_BLK = 8


def _vq_tile_kernel(z_ref, tab_ref, tsq_ref, zq_ref, idx_ref, dmin_ref):
    z = z_ref[0]
    table = tab_ref[...]
    s = z.shape[1]

    m2 = jax.lax.dot_general(
        table + table, z, (((1,), (0,)), ((), ())),
        preferred_element_type=jnp.float32,
    )

    t_sq = tsq_ref[...]
    z_sq = jnp.sum(z * z, axis=0)
    z_sq_row = z_sq[None, :]

    n_blocks = _NUM_CODES // _BLK
    minval = None
    minblk = None
    for k in range(n_blocks):
        a = z_sq_row + t_sq[k * _BLK:(k + 1) * _BLK, :]
        d_blk = a - m2[k * _BLK:(k + 1) * _BLK, :]
        if k == 0:
            minval = d_blk
            minblk = jnp.zeros((_BLK, s), jnp.int32)
        else:
            lt = d_blk < minval
            minval = jnp.where(lt, d_blk, minval)
            minblk = jnp.where(lt, k, minblk)

    dmin = jnp.min(minval, axis=0)
    sub_iota = jax.lax.broadcasted_iota(jnp.int32, (_BLK, s), 0)
    code = minblk * _BLK + sub_iota
    cand = jnp.where(minval == dmin[None, :], code, _NUM_CODES)
    idx = jnp.min(cand, axis=0)

    iota = jax.lax.broadcasted_iota(jnp.int32, (_NUM_CODES, s), 0)
    ohb = (iota == idx[None, :]).astype(jnp.bfloat16)
    t_hi = table.astype(jnp.bfloat16)
    t_lo = (table - t_hi.astype(jnp.float32)).astype(jnp.bfloat16)
    zq = jax.lax.dot_general(
        t_hi, ohb, (((0,), (0,)), ((), ())),
        preferred_element_type=jnp.float32,
    ) + jax.lax.dot_general(
        t_lo, ohb, (((0,), (0,)), ((), ())),
        preferred_element_type=jnp.float32,
    )

    zq_ref[0] = zq
    idx_ref[0, 0, 0] = idx
    dmin_ref[0, 0, 0] = dmin


def kernel(z, table):
    b, emb, d_, h, w = z.shape
    spatial = d_ * h * w
    z3 = z.reshape(b, emb, spatial)
    ns = spatial // _S_TILE

    zq3, idx4, dmin4 = pl.pallas_call(
        _vq_tile_kernel,
        grid=(b, ns),
        in_specs=[
            pl.BlockSpec((1, emb, _S_TILE), lambda i, j: (i, 0, j)),
            pl.BlockSpec((_NUM_CODES, emb), lambda i, j: (0, 0)),
            pl.BlockSpec((_NUM_CODES, 1), lambda i, j: (0, 0)),
        ],
        out_specs=[
            pl.BlockSpec((1, emb, _S_TILE), lambda i, j: (i, 0, j)),
            pl.BlockSpec((1, 1, 1, _S_TILE), lambda i, j: (i, j, 0, 0)),
            pl.BlockSpec((1, 1, 1, _S_TILE), lambda i, j: (i, j, 0, 0)),
        ],
        out_shape=[
            jax.ShapeDtypeStruct((b, emb, spatial), jnp.float32),
            jax.ShapeDtypeStruct((b, ns, 1, _S_TILE), jnp.int32),
            jax.ShapeDtypeStruct((b, ns, 1, _S_TILE), jnp.float32),
        ],
        compiler_params=pltpu.CompilerParams(
            dimension_semantics=("parallel", "parallel")),
    )(z3, table, jnp.sum(table ** 2, axis=1)[:, None])

    z_q_st = zq3.reshape(b, emb, d_, h, w)
    indices = idx4.reshape(b * spatial)
    n_elems = b * emb * spatial
    vq_loss = jnp.sum(dmin4) * jnp.float32(1.25 / n_elems)
    return (z_q_st, vq_loss, indices)


# speedup vs baseline: 1.1951x; 1.1951x over previous
import jax
import jax.numpy as jnp
from jax.experimental import pallas as pl
from jax.experimental.pallas import tpu as pltpu

_NUM_CODES = 1024
_EMB = 64
_S_TILE = 4096
_BLK = 16


def _vq_tile_kernel(z_ref, tab_ref, tsq_ref, zq_ref, idx_ref, loss_ref):
    z = z_ref[0]
    table = tab_ref[...]
    s = z.shape[1]

    m2 = jax.lax.dot_general(
        table + table, z, (((1,), (0,)), ((), ())),
        preferred_element_type=jnp.float32,
    )

    t_sq = tsq_ref[...]
    z_sq = jnp.sum(z * z, axis=0)
    z_sq_row = z_sq[None, :]

    n_blocks = _NUM_CODES // _BLK
    minval = None
    minblk = None
    for k in range(n_blocks):
        a = z_sq_row + t_sq[k * _BLK:(k + 1) * _BLK, :]
        d_blk = a - m2[k * _BLK:(k + 1) * _BLK, :]
        if k == 0:
            minval = d_blk
            minblk = jnp.zeros((_BLK, s), jnp.int32)
        else:
            lt = d_blk < minval
            minval = jnp.where(lt, d_blk, minval)
            minblk = jnp.where(lt, k, minblk)

    dmin = jnp.min(minval, axis=0)
    sub_iota = jax.lax.broadcasted_iota(jnp.int32, (_BLK, s), 0)
    code = minblk * _BLK + sub_iota
    cand = jnp.where(minval == dmin[None, :], code, _NUM_CODES)
    idx = jnp.min(cand, axis=0)

    iota = jax.lax.broadcasted_iota(jnp.int32, (_NUM_CODES, s), 0)
    oh = (iota == idx[None, :]).astype(jnp.float32)
    zq = jax.lax.dot_general(
        table, oh, (((0,), (0,)), ((), ())),
        preferred_element_type=jnp.float32,
    )

    zq_ref[0] = zq
    idx_ref[0, 0, 0] = idx
    loss_ref[0, 0, 0, 0] = jnp.sum(dmin)


def kernel(z, table):
    b, emb, d_, h, w = z.shape
    spatial = d_ * h * w
    z3 = z.reshape(b, emb, spatial)
    ns = spatial // _S_TILE

    zq3, idx4, loss4 = pl.pallas_call(
        _vq_tile_kernel,
        grid=(b, ns),
        in_specs=[
            pl.BlockSpec((1, emb, _S_TILE), lambda i, j: (i, 0, j)),
            pl.BlockSpec((_NUM_CODES, emb), lambda i, j: (0, 0)),
            pl.BlockSpec((_NUM_CODES, 1), lambda i, j: (0, 0)),
        ],
        out_specs=[
            pl.BlockSpec((1, emb, _S_TILE), lambda i, j: (i, 0, j)),
            pl.BlockSpec((1, 1, 1, _S_TILE), lambda i, j: (i, j, 0, 0)),
            pl.BlockSpec((1, 1, 1, 1), lambda i, j: (i, j, 0, 0),
                         memory_space=pltpu.SMEM),
        ],
        out_shape=[
            jax.ShapeDtypeStruct((b, emb, spatial), jnp.float32),
            jax.ShapeDtypeStruct((b, ns, 1, _S_TILE), jnp.int32),
            jax.ShapeDtypeStruct((b, ns, 1, 1), jnp.float32),
        ],
    )(z3, table, jnp.sum(table ** 2, axis=1)[:, None])

    z_q_st = zq3.reshape(b, emb, d_, h, w)
    indices = idx4.reshape(b * spatial)
    n_elems = b * emb * spatial
    vq_loss = jnp.sum(loss4) * jnp.float32(1.25 / n_elems)
    return (z_q_st, vq_loss, indices)


# speedup vs baseline: 1.2012x; 1.0051x over previous
import jax
import jax.numpy as jnp
from jax.experimental import pallas as pl
from jax.experimental.pallas import tpu as pltpu

_NUM_CODES = 1024
_EMB = 64
_S_TILE = 4096
_BLK = 8


def _vq_tile_kernel(z_ref, tab_ref, tsq_ref, zq_ref, idx_ref, loss_ref):
    z = z_ref[0]
    table = tab_ref[...]
    s = z.shape[1]

    m2 = jax.lax.dot_general(
        table + table, z, (((1,), (0,)), ((), ())),
        preferred_element_type=jnp.float32,
    )

    t_sq = tsq_ref[...]
    z_sq = jnp.sum(z * z, axis=0)
    z_sq_row = z_sq[None, :]

    n_blocks = _NUM_CODES // _BLK
    minval = None
    minblk = None
    for k in range(n_blocks):
        a = z_sq_row + t_sq[k * _BLK:(k + 1) * _BLK, :]
        d_blk = a - m2[k * _BLK:(k + 1) * _BLK, :]
        if k == 0:
            minval = d_blk
            minblk = jnp.zeros((_BLK, s), jnp.int32)
        else:
            lt = d_blk < minval
            minval = jnp.where(lt, d_blk, minval)
            minblk = jnp.where(lt, k, minblk)

    dmin = jnp.min(minval, axis=0)
    sub_iota = jax.lax.broadcasted_iota(jnp.int32, (_BLK, s), 0)
    code = minblk * _BLK + sub_iota
    cand = jnp.where(minval == dmin[None, :], code, _NUM_CODES)
    idx = jnp.min(cand, axis=0)

    iota = jax.lax.broadcasted_iota(jnp.int32, (_NUM_CODES, s), 0)
    oh = (iota == idx[None, :]).astype(jnp.float32)
    zq = jax.lax.dot_general(
        table, oh, (((0,), (0,)), ((), ())),
        preferred_element_type=jnp.float32,
    )

    zq_ref[0] = zq
    idx_ref[0, 0, 0] = idx
    loss_ref[0, 0, 0, 0] = jnp.sum(dmin)


def kernel(z, table):
    b, emb, d_, h, w = z.shape
    spatial = d_ * h * w
    z3 = z.reshape(b, emb, spatial)
    ns = spatial // _S_TILE

    zq3, idx4, loss4 = pl.pallas_call(
        _vq_tile_kernel,
        grid=(b, ns),
        in_specs=[
            pl.BlockSpec((1, emb, _S_TILE), lambda i, j: (i, 0, j)),
            pl.BlockSpec((_NUM_CODES, emb), lambda i, j: (0, 0)),
            pl.BlockSpec((_NUM_CODES, 1), lambda i, j: (0, 0)),
        ],
        out_specs=[
            pl.BlockSpec((1, emb, _S_TILE), lambda i, j: (i, 0, j)),
            pl.BlockSpec((1, 1, 1, _S_TILE), lambda i, j: (i, j, 0, 0)),
            pl.BlockSpec((1, 1, 1, 1), lambda i, j: (i, j, 0, 0),
                         memory_space=pltpu.SMEM),
        ],
        out_shape=[
            jax.ShapeDtypeStruct((b, emb, spatial), jnp.float32),
            jax.ShapeDtypeStruct((b, ns, 1, _S_TILE), jnp.int32),
            jax.ShapeDtypeStruct((b, ns, 1, 1), jnp.float32),
        ],
    )(z3, table, jnp.sum(table ** 2, axis=1)[:, None])

    z_q_st = zq3.reshape(b, emb, d_, h, w)
    indices = idx4.reshape(b * spatial)
    n_elems = b * emb * spatial
    vq_loss = jnp.sum(loss4) * jnp.float32(1.25 / n_elems)
    return (z_q_st, vq_loss, indices)
